# baseline (speedup 1.0000x reference)
import functools

import jax
import jax.numpy as jnp
from jax import lax
from jax.experimental import pallas as pl
from jax.experimental.pallas import tpu as pltpu
from jax.experimental.pallas import tpu_sc as plsc

NC = 2
NS = 16
NW = NC * NS
LANES = 16
K = 32
C = 400
BW = 32768
Q = BW // 4


def _detile_body(x_ref, o_ref):
    x = x_ref[...]
    eye = jnp.eye(K, dtype=jnp.float32)
    parts = [
        lax.dot_general(x[:, m * Q:(m + 1) * Q], eye,
                        (((0,), (0,)), ((), ())),
                        precision=lax.Precision.HIGHEST,
                        preferred_element_type=jnp.float32)
        for m in range(4)
    ]
    o_ref[...] = jnp.concatenate(parts, axis=1)


def _detile(table):
    tt = table.T
    v = table.shape[0]
    grid = pl.cdiv(v, BW)
    out = pl.pallas_call(
        _detile_body,
        grid=(grid,),
        in_specs=[pl.BlockSpec((K, BW), lambda j: (0, j))],
        out_specs=pl.BlockSpec((Q, 128), lambda j: (j, 0)),
        out_shape=jax.ShapeDtypeStruct((grid * Q, 128), jnp.float32),
    )(tt)
    return out.reshape(grid * BW, K)


_QSHIFT = Q.bit_length() - 1


def _perm(i):
    return (
        jnp.bitwise_and(i, ~jnp.int32(BW - 1))
        | (jnp.bitwise_and(i, Q - 1) << 2)
        | (jnp.bitwise_and(i, BW - 1) >> _QSHIFT)
    )


def _fpmc_body(users_hbm, prev_hbm, items_hbm, bidx_hbm, ue_hbm, ie_hbm,
               pe_hbm, ne_hbm, ib_hbm, out_hbm,
               u_idx, p_idx, i_idx, b_idx, ui_v, ii_v, ip_v, ic_v, ib_v,
               out_v, isem0, isem1, gsem0, gsem1, osem0, osem1, *, n_per_w):
    nchunks = n_per_w // C
    wid = lax.axis_index("s") * NC + lax.axis_index("c")
    wbase = wid * n_per_w
    isems = (isem0, isem1)
    gsems = (gsem0, gsem1)
    osems = (osem0, osem1)

    def idx_copies(cur, slot):
        base = wbase + cur * C
        sem = isems[slot]
        return [
            pltpu.make_async_copy(users_hbm.at[pl.ds(base, C)], u_idx.at[slot], sem),
            pltpu.make_async_copy(prev_hbm.at[pl.ds(base, C)], p_idx.at[slot], sem),
            pltpu.make_async_copy(items_hbm.at[pl.ds(base, C)], i_idx.at[slot], sem),
            pltpu.make_async_copy(bidx_hbm.at[pl.ds(base, C)], b_idx.at[slot], sem),
        ]

    def gather_copies(slot):
        sem = gsems[slot]
        return [
            pltpu.make_async_copy(ue_hbm.at[u_idx.at[slot]], ui_v.at[slot], sem),
            pltpu.make_async_copy(ie_hbm.at[i_idx.at[slot]], ii_v.at[slot], sem),
            pltpu.make_async_copy(pe_hbm.at[p_idx.at[slot]], ip_v.at[slot], sem),
            pltpu.make_async_copy(ne_hbm.at[i_idx.at[slot]], ic_v.at[slot], sem),
            pltpu.make_async_copy(ib_hbm.at[b_idx.at[slot]], ib_v.at[slot], sem),
        ]

    def out_copy(cur, slot):
        base = wbase + cur * C
        return pltpu.make_async_copy(out_v.at[slot], out_hbm.at[pl.ds(base, C)],
                                     osems[slot])

    def compute(slot):
        lane = lax.iota(jnp.int32, LANES)
        uir = ui_v.at[slot]
        iir = ii_v.at[slot]
        ipr = ip_v.at[slot]
        icr = ic_v.at[slot]
        ibr = ib_v.at[slot]
        outr = out_v.at[slot]

        def group(g, carry):
            ri = lane + g * LANES
            acc = ibr[pl.ds(g * LANES, LANES)]
            for k in range(K):
                ck = jnp.bitwise_and(lane + k, K - 1)
                a = plsc.load_gather(uir, [ri, ck])
                b = plsc.load_gather(iir, [ri, ck])
                c = plsc.load_gather(ipr, [ri, ck])
                d = plsc.load_gather(icr, [ri, ck])
                acc = acc + a * b + c * d
            outr[pl.ds(g * LANES, LANES)] = acc
            return carry

        lax.fori_loop(0, C // LANES, group, 0)

    for cp in idx_copies(0, 0):
        cp.start()
    for cp in idx_copies(1, 1):
        cp.start()
    for cp in idx_copies(0, 0):
        cp.wait()
    for cp in gather_copies(0):
        cp.start()

    def chunk_pair(i, carry):
        for s in (0, 1):
            cur = 2 * i + s
            for cp in gather_copies(s):
                cp.wait()

            @pl.when(cur + 2 < nchunks)
            def _():
                for cp in idx_copies(cur + 2, s):
                    cp.start()

            @pl.when(cur + 1 < nchunks)
            def _():
                for cp in idx_copies(cur + 1, 1 - s):
                    cp.wait()
                for cp in gather_copies(1 - s):
                    cp.start()

            @pl.when(cur >= 2)
            def _():
                out_copy(cur - 2, s).wait()

            compute(s)
            out_copy(cur, s).start()
        return carry

    lax.fori_loop(0, nchunks // 2, chunk_pair, 0)
    out_copy(nchunks - 2, 0).wait()
    out_copy(nchunks - 1, 1).wait()


def kernel(users, prev, items, user_embs, item_embs, prev_embs, next_embs,
           item_bias):
    B, L = users.shape
    BL = B * L
    assert BL % (NW * C) == 0
    n_per_w = BL // NW

    ue = _detile(user_embs)
    ie = _detile(item_embs)
    pe = _detile(prev_embs)
    ne = _detile(next_embs)

    grid_kernel = functools.partial(
        pl.kernel,
        out_type=jax.ShapeDtypeStruct((BL,), jnp.float32),
        mesh=plsc.VectorSubcoreMesh(core_axis_name="c", subcore_axis_name="s"),
        scratch_types=[
            pltpu.VMEM((2, C), jnp.int32),
            pltpu.VMEM((2, C), jnp.int32),
            pltpu.VMEM((2, C), jnp.int32),
            pltpu.VMEM((2, C), jnp.int32),
            pltpu.VMEM((2, C, K), jnp.float32),
            pltpu.VMEM((2, C, K), jnp.float32),
            pltpu.VMEM((2, C, K), jnp.float32),
            pltpu.VMEM((2, C, K), jnp.float32),
            pltpu.VMEM((2, C), jnp.float32),
            pltpu.VMEM((2, C), jnp.float32),
            pltpu.SemaphoreType.DMA,
            pltpu.SemaphoreType.DMA,
            pltpu.SemaphoreType.DMA,
            pltpu.SemaphoreType.DMA,
            pltpu.SemaphoreType.DMA,
            pltpu.SemaphoreType.DMA,
        ],
        compiler_params=pltpu.CompilerParams(
            needs_layout_passes=False, use_tc_tiling_on_sc=False),
    )
    fpmc = grid_kernel(functools.partial(_fpmc_body, n_per_w=n_per_w))
    uf = _perm(users.reshape(BL))
    pf = _perm(prev.reshape(BL))
    itf = _perm(items.reshape(BL))
    ibf = jnp.bitwise_and(items.reshape(BL), jnp.int32(0x7FFFFFFF))
    out = fpmc(uf, pf, itf, ibf, ue, ie, pe, ne, item_bias.reshape(-1))
    out2d = out.reshape(B, L)
    return jnp.where(users >= 0, out2d, jnp.float32(0.0))


# speedup vs baseline: 1.8013x; 1.8013x over previous
import functools

import jax
import jax.numpy as jnp
from jax import lax
from jax.experimental import pallas as pl
from jax.experimental.pallas import tpu as pltpu
from jax.experimental.pallas import tpu_sc as plsc

NC = 2
NS = 16
NW = NC * NS
LANES = 16
K = 32
C = 400
BW = 32768
Q = BW // 4


def _detile_body(x_ref, o_ref):
    x = x_ref[...]
    eye = jnp.eye(K, dtype=jnp.float32)
    parts = [
        lax.dot_general(x[:, m * Q:(m + 1) * Q], eye,
                        (((0,), (0,)), ((), ())),
                        preferred_element_type=jnp.float32)
        for m in range(4)
    ]
    o_ref[...] = jnp.concatenate(parts, axis=1)


def _detile(table):
    tt = table.T
    v = table.shape[0]
    grid = pl.cdiv(v, BW)
    out = pl.pallas_call(
        _detile_body,
        grid=(grid,),
        in_specs=[pl.BlockSpec((K, BW), lambda j: (0, j))],
        out_specs=pl.BlockSpec((Q, 128), lambda j: (j, 0)),
        out_shape=jax.ShapeDtypeStruct((grid * Q, 128), jnp.float32),
    )(tt)
    return out.reshape(grid * BW, K)


_QSHIFT = Q.bit_length() - 1


def _perm(i):
    return (
        jnp.bitwise_and(i, ~jnp.int32(BW - 1))
        | (jnp.bitwise_and(i, Q - 1) << 2)
        | (jnp.bitwise_and(i, BW - 1) >> _QSHIFT)
    )


def _fpmc_body(users_hbm, prev_hbm, items_hbm, bidx_hbm, ue_hbm, ie_hbm,
               pe_hbm, ne_hbm, ib_hbm, out_hbm,
               u_idx, p_idx, i_idx, b_idx, ui_v, ii_v, ip_v, ic_v, ib_v,
               out_v, isem0, isem1, gsem0, gsem1, osem0, osem1, *, n_per_w):
    nchunks = n_per_w // C
    wid = lax.axis_index("s") * NC + lax.axis_index("c")
    wbase = wid * n_per_w
    isems = (isem0, isem1)
    gsems = (gsem0, gsem1)
    osems = (osem0, osem1)

    def idx_copies(cur, slot):
        base = wbase + cur * C
        sem = isems[slot]
        return [
            pltpu.make_async_copy(users_hbm.at[pl.ds(base, C)], u_idx.at[slot], sem),
            pltpu.make_async_copy(prev_hbm.at[pl.ds(base, C)], p_idx.at[slot], sem),
            pltpu.make_async_copy(items_hbm.at[pl.ds(base, C)], i_idx.at[slot], sem),
            pltpu.make_async_copy(bidx_hbm.at[pl.ds(base, C)], b_idx.at[slot], sem),
        ]

    def gather_copies(slot):
        sem = gsems[slot]
        return [
            pltpu.make_async_copy(ue_hbm.at[u_idx.at[slot]], ui_v.at[slot], sem),
            pltpu.make_async_copy(ie_hbm.at[i_idx.at[slot]], ii_v.at[slot], sem),
            pltpu.make_async_copy(pe_hbm.at[p_idx.at[slot]], ip_v.at[slot], sem),
            pltpu.make_async_copy(ne_hbm.at[i_idx.at[slot]], ic_v.at[slot], sem),
            pltpu.make_async_copy(ib_hbm.at[b_idx.at[slot]], ib_v.at[slot], sem),
        ]

    def out_copy(cur, slot):
        base = wbase + cur * C
        return pltpu.make_async_copy(out_v.at[slot], out_hbm.at[pl.ds(base, C)],
                                     osems[slot])

    def compute(slot):
        lane = lax.iota(jnp.int32, LANES)
        uir = ui_v.at[slot]
        iir = ii_v.at[slot]
        ipr = ip_v.at[slot]
        icr = ic_v.at[slot]
        ibr = ib_v.at[slot]
        outr = out_v.at[slot]

        def group(g, carry):
            ri = lane + g * LANES
            acc = ibr[pl.ds(g * LANES, LANES)]
            for k in range(K):
                ck = jnp.bitwise_and(lane + k, K - 1)
                a = plsc.load_gather(uir, [ri, ck])
                b = plsc.load_gather(iir, [ri, ck])
                c = plsc.load_gather(ipr, [ri, ck])
                d = plsc.load_gather(icr, [ri, ck])
                acc = acc + a * b + c * d
            outr[pl.ds(g * LANES, LANES)] = acc
            return carry

        lax.fori_loop(0, C // LANES, group, 0)

    for cp in idx_copies(0, 0):
        cp.start()
    for cp in idx_copies(1, 1):
        cp.start()
    for cp in idx_copies(0, 0):
        cp.wait()
    for cp in gather_copies(0):
        cp.start()

    def chunk_pair(i, carry):
        for s in (0, 1):
            cur = 2 * i + s
            for cp in gather_copies(s):
                cp.wait()

            @pl.when(cur + 2 < nchunks)
            def _():
                for cp in idx_copies(cur + 2, s):
                    cp.start()

            @pl.when(cur + 1 < nchunks)
            def _():
                for cp in idx_copies(cur + 1, 1 - s):
                    cp.wait()
                for cp in gather_copies(1 - s):
                    cp.start()

            @pl.when(cur >= 2)
            def _():
                out_copy(cur - 2, s).wait()

            compute(s)
            out_copy(cur, s).start()
        return carry

    lax.fori_loop(0, nchunks // 2, chunk_pair, 0)
    out_copy(nchunks - 2, 0).wait()
    out_copy(nchunks - 1, 1).wait()


def kernel(users, prev, items, user_embs, item_embs, prev_embs, next_embs,
           item_bias):
    B, L = users.shape
    BL = B * L
    assert BL % (NW * C) == 0
    n_per_w = BL // NW

    ue = _detile(user_embs)
    ie = _detile(item_embs)
    pe = _detile(prev_embs)
    ne = _detile(next_embs)

    grid_kernel = functools.partial(
        pl.kernel,
        out_type=jax.ShapeDtypeStruct((BL,), jnp.float32),
        mesh=plsc.VectorSubcoreMesh(core_axis_name="c", subcore_axis_name="s"),
        scratch_types=[
            pltpu.VMEM((2, C), jnp.int32),
            pltpu.VMEM((2, C), jnp.int32),
            pltpu.VMEM((2, C), jnp.int32),
            pltpu.VMEM((2, C), jnp.int32),
            pltpu.VMEM((2, C, K), jnp.float32),
            pltpu.VMEM((2, C, K), jnp.float32),
            pltpu.VMEM((2, C, K), jnp.float32),
            pltpu.VMEM((2, C, K), jnp.float32),
            pltpu.VMEM((2, C), jnp.float32),
            pltpu.VMEM((2, C), jnp.float32),
            pltpu.SemaphoreType.DMA,
            pltpu.SemaphoreType.DMA,
            pltpu.SemaphoreType.DMA,
            pltpu.SemaphoreType.DMA,
            pltpu.SemaphoreType.DMA,
            pltpu.SemaphoreType.DMA,
        ],
        compiler_params=pltpu.CompilerParams(
            needs_layout_passes=False, use_tc_tiling_on_sc=False),
    )
    fpmc = grid_kernel(functools.partial(_fpmc_body, n_per_w=n_per_w))
    uf = _perm(users.reshape(BL))
    pf = _perm(prev.reshape(BL))
    itf = _perm(items.reshape(BL))
    ibf = jnp.bitwise_and(items.reshape(BL), jnp.int32(0x7FFFFFFF))
    out = fpmc(uf, pf, itf, ibf, ue, ie, pe, ne, item_bias.reshape(-1))
    out2d = out.reshape(B, L)
    return jnp.where(users >= 0, out2d, jnp.float32(0.0))
